# baseline (speedup 1.0000x reference)
import functools

import jax
import jax.numpy as jnp
from jax import lax
from jax.experimental import pallas as pl
from jax.experimental.pallas import tpu as pltpu
from jax.experimental.pallas import tpu_sc as plsc

IN_F = 4096
OUT_F = 4096
N_W = IN_F * OUT_F
RANK = 15938354
NC, NS, L = 2, 16, 16
NW = NC * NS
PER_W = N_W // NW
ROWS_W = OUT_F // NW
CROWS1 = 8
NCH1 = ROWS_W // CROWS1
CROWS2 = 4
NCH2 = ROWS_W // CROWS2
VPC = IN_F // L
UNROLL = 8
B1 = 16384
B2 = 65536

def _mesh():
    return plsc.VectorSubcoreMesh(
        core_axis_name="c", subcore_axis_name="s",
        num_cores=NC, num_subcores=NS,
    )


def _worker_id():
    return lax.axis_index("s") * NC + lax.axis_index("c")


def _zero_hist(hist, nbins):
    z = jnp.zeros((L,), jnp.int32)

    def body(i, _):
        for u in range(UNROLL):
            hist[pl.ds((i * UNROLL + u) * L, L)] = z
        return 0

    lax.fori_loop(0, nbins // (L * UNROLL), body, 0)


@functools.cache
def _hist_pass1_kernel():
    return pl.kernel(
        _hist_pass1_body,
        out_type=jax.ShapeDtypeStruct((NW, B1), jnp.int32),
        mesh=_mesh(),
        scratch_types=[
            pltpu.VMEM((CROWS1, IN_F), jnp.float32),
            pltpu.VMEM((CROWS1, IN_F), jnp.float32),
            pltpu.VMEM((B1,), jnp.int32),
            pltpu.SemaphoreType.DMA,
            pltpu.SemaphoreType.DMA,
        ],
        compiler_params=pltpu.CompilerParams(needs_layout_passes=False),
    )


def _hist_pass1_body(w_hbm, out_hbm, buf_a, buf_b, hist, sem_a, sem_b):
    wid = _worker_id()
    row0 = wid * ROWS_W
    _zero_hist(hist, B1)
    ones = jnp.ones((L,), jnp.int32)

    def start(g, buf, sem):
        g = jnp.minimum(g, NCH1 - 1)
        pltpu.async_copy(w_hbm.at[pl.ds(row0 + g * CROWS1, CROWS1), :], buf, sem)

    def process(buf):
        @plsc.parallel_loop(0, CROWS1 * VPC, unroll=UNROLL)
        def vec_body(j):
            r = lax.shift_right_logical(j, 8)
            c = j & (VPC - 1)
            p = plsc.bitcast(jnp.abs(buf[r, pl.ds(c * L, L)]), jnp.int32)
            b = jnp.minimum(lax.shift_right_logical(p, 16), B1 - 1)
            plsc.addupdate_scatter(hist, [b], ones)

    start(0, buf_a, sem_a)

    def pair_body(g2, _):
        start(2 * g2 + 1, buf_b, sem_b)
        pltpu.make_async_copy(w_hbm.at[pl.ds(0, CROWS1), :], buf_a, sem_a).wait()
        process(buf_a)
        start(2 * g2 + 2, buf_a, sem_a)
        pltpu.make_async_copy(w_hbm.at[pl.ds(0, CROWS1), :], buf_b, sem_b).wait()
        process(buf_b)
        return 0

    lax.fori_loop(0, NCH1 // 2, pair_body, 0)
    pltpu.make_async_copy(w_hbm.at[pl.ds(0, CROWS1), :], buf_a, sem_a).wait()
    pltpu.sync_copy(hist, out_hbm.at[wid])


@functools.cache
def _hist_pass2_kernel():
    return pl.kernel(
        _hist_pass2_body,
        out_type=jax.ShapeDtypeStruct((NW, B2), jnp.int32),
        mesh=_mesh(),
        scratch_types=[
            pltpu.VMEM((CROWS2, IN_F), jnp.float32),
            pltpu.VMEM((CROWS2, IN_F), jnp.float32),
            pltpu.VMEM((B2,), jnp.int32),
            pltpu.VMEM((L,), jnp.int32),
            pltpu.SemaphoreType.DMA,
            pltpu.SemaphoreType.DMA,
        ],
        compiler_params=pltpu.CompilerParams(needs_layout_passes=False),
    )


def _hist_pass2_body(w_hbm, b0_hbm, out_hbm, buf_a, buf_b, hist, b0_v,
                     sem_a, sem_b):
    wid = _worker_id()
    row0 = wid * ROWS_W
    _zero_hist(hist, B2)
    pltpu.sync_copy(b0_hbm, b0_v)
    b0 = b0_v[...]
    ones = jnp.ones((L,), jnp.int32)

    def start(g, buf, sem):
        g = jnp.minimum(g, NCH2 - 1)
        pltpu.async_copy(w_hbm.at[pl.ds(row0 + g * CROWS2, CROWS2), :], buf, sem)

    def process(buf):
        @plsc.parallel_loop(0, CROWS2 * VPC, unroll=UNROLL)
        def vec_body(j):
            r = lax.shift_right_logical(j, 8)
            c = j & (VPC - 1)
            p = plsc.bitcast(jnp.abs(buf[r, pl.ds(c * L, L)]), jnp.int32)
            top = lax.shift_right_logical(p, 16)
            low = p & 0xFFFF
            plsc.addupdate_scatter(hist, [low], ones, mask=top == b0)

    start(0, buf_a, sem_a)

    def pair_body(g2, _):
        start(2 * g2 + 1, buf_b, sem_b)
        pltpu.make_async_copy(w_hbm.at[pl.ds(0, CROWS2), :], buf_a, sem_a).wait()
        process(buf_a)
        start(2 * g2 + 2, buf_a, sem_a)
        pltpu.make_async_copy(w_hbm.at[pl.ds(0, CROWS2), :], buf_b, sem_b).wait()
        process(buf_b)
        return 0

    lax.fori_loop(0, NCH2 // 2, pair_body, 0)
    pltpu.make_async_copy(w_hbm.at[pl.ds(0, CROWS2), :], buf_a, sem_a).wait()
    pltpu.sync_copy(hist, out_hbm.at[wid])


BM = 2048
BN = 512


def _premask_body(t_ref, w_ref, o_ref):
    w = w_ref[...]
    o_ref[...] = jnp.where(jnp.abs(w) >= t_ref[0, 0], w, 0.0).astype(
        jnp.bfloat16)


def _premask(weight, threshold):
    blk = 512
    return pl.pallas_call(
        _premask_body,
        grid=(OUT_F // blk,),
        in_specs=[
            pl.BlockSpec(memory_space=pltpu.SMEM),
            pl.BlockSpec((blk, IN_F), lambda i: (i, 0)),
        ],
        out_specs=pl.BlockSpec((blk, IN_F), lambda i: (i, 0)),
        out_shape=jax.ShapeDtypeStruct((OUT_F, IN_F), jnp.bfloat16),
        compiler_params=pltpu.CompilerParams(
            dimension_semantics=("parallel",),
        ),
    )(threshold.reshape(1, 1), weight)


def _mm_body(x_ref, w_ref, b_ref, o_ref):
    acc = lax.dot_general(
        x_ref[...], w_ref[...], (((1,), (1,)), ((), ())),
        preferred_element_type=jnp.float32,
    )
    o_ref[...] = acc + b_ref[...]


def _masked_matmul(inputs, weight, bias, threshold):
    m_tokens = inputs.shape[0]
    w_eff = _premask(weight, threshold)
    grid = (m_tokens // BM, OUT_F // BN)
    return pl.pallas_call(
        _mm_body,
        grid=grid,
        in_specs=[
            pl.BlockSpec((BM, IN_F), lambda m, n: (m, 0)),
            pl.BlockSpec((BN, IN_F), lambda m, n: (n, 0)),
            pl.BlockSpec((1, BN), lambda m, n: (0, n)),
        ],
        out_specs=pl.BlockSpec((BM, BN), lambda m, n: (m, n)),
        out_shape=jax.ShapeDtypeStruct((m_tokens, OUT_F), jnp.float32),
        compiler_params=pltpu.CompilerParams(
            dimension_semantics=("parallel", "parallel"),
            vmem_limit_bytes=100 * 1024 * 1024,
        ),
    )(inputs.astype(jnp.bfloat16), w_eff, bias.reshape(1, OUT_F))


def _hist_pass2(w_flat, b0_vec):
    return _hist_pass2_kernel()(w_flat, b0_vec)


def _threshold_from_hists(w_flat, h1_rows):
    h1 = jnp.sum(h1_rows, axis=0)
    c1 = jnp.cumsum(h1)
    b0 = jnp.searchsorted(c1, RANK, side="right").astype(jnp.int32)
    below = jnp.where(b0 > 0, c1[jnp.maximum(b0 - 1, 0)], 0)
    r0 = RANK - below
    h2_rows = _hist_pass2(w_flat, jnp.full((L,), b0, jnp.int32))
    h2 = jnp.sum(h2_rows, axis=0)
    c2 = jnp.cumsum(h2)
    low = jnp.searchsorted(c2, r0, side="right").astype(jnp.int32)
    pattern = (b0 << 16) | low
    return lax.bitcast_convert_type(pattern, jnp.float32)


def kernel(inputs, weight, bias):
    h1_rows = _hist_pass1_kernel()(weight)
    threshold = _threshold_from_hists(weight, h1_rows)
    return _masked_matmul(inputs, weight, bias, threshold)


# speedup vs baseline: 1.0541x; 1.0541x over previous
import functools

import jax
import jax.numpy as jnp
from jax import lax
from jax.experimental import pallas as pl
from jax.experimental.pallas import tpu as pltpu
from jax.experimental.pallas import tpu_sc as plsc

IN_F = 4096
OUT_F = 4096
N_W = IN_F * OUT_F
RANK = 15938354
NC, NS, L = 2, 16, 16
NW = NC * NS
PER_W = N_W // NW
ROWS_W = OUT_F // NW
CROWS1 = 8
NCH1 = ROWS_W // CROWS1
CROWS2 = 4
NCH2 = ROWS_W // CROWS2
VPC = IN_F // L
UNROLL = 16
B1 = 16384
B2 = 65536

def _mesh():
    return plsc.VectorSubcoreMesh(
        core_axis_name="c", subcore_axis_name="s",
        num_cores=NC, num_subcores=NS,
    )


def _worker_id():
    return lax.axis_index("s") * NC + lax.axis_index("c")


def _zero_hist(hist, nbins):
    z = jnp.zeros((L,), jnp.int32)

    def body(i, _):
        for u in range(UNROLL):
            hist[pl.ds((i * UNROLL + u) * L, L)] = z
        return 0

    lax.fori_loop(0, nbins // (L * UNROLL), body, 0)


@functools.cache
def _hist_pass1_kernel():
    return pl.kernel(
        _hist_pass1_body,
        out_type=jax.ShapeDtypeStruct((NW, B1), jnp.int32),
        mesh=_mesh(),
        scratch_types=[
            pltpu.VMEM((CROWS1, IN_F), jnp.float32),
            pltpu.VMEM((CROWS1, IN_F), jnp.float32),
            pltpu.VMEM((B1,), jnp.int32),
            pltpu.SemaphoreType.DMA,
            pltpu.SemaphoreType.DMA,
        ],
        compiler_params=pltpu.CompilerParams(needs_layout_passes=False),
    )


def _hist_pass1_body(w_hbm, out_hbm, buf_a, buf_b, hist, sem_a, sem_b):
    wid = _worker_id()
    row0 = wid * ROWS_W
    _zero_hist(hist, B1)
    ones = jnp.ones((L,), jnp.int32)

    def start(g, buf, sem):
        g = jnp.minimum(g, NCH1 - 1)
        pltpu.async_copy(w_hbm.at[pl.ds(row0 + g * CROWS1, CROWS1), :], buf, sem)

    def process(buf):
        @plsc.parallel_loop(0, CROWS1 * VPC, unroll=UNROLL)
        def vec_body(j):
            r = lax.shift_right_logical(j, 8)
            c = j & (VPC - 1)
            p = plsc.bitcast(jnp.abs(buf[r, pl.ds(c * L, L)]), jnp.int32)
            b = jnp.minimum(lax.shift_right_logical(p, 16), B1 - 1)
            plsc.addupdate_scatter(hist, [b], ones)

    start(0, buf_a, sem_a)

    def pair_body(g2, _):
        start(2 * g2 + 1, buf_b, sem_b)
        pltpu.make_async_copy(w_hbm.at[pl.ds(0, CROWS1), :], buf_a, sem_a).wait()
        process(buf_a)
        start(2 * g2 + 2, buf_a, sem_a)
        pltpu.make_async_copy(w_hbm.at[pl.ds(0, CROWS1), :], buf_b, sem_b).wait()
        process(buf_b)
        return 0

    lax.fori_loop(0, NCH1 // 2, pair_body, 0)
    pltpu.make_async_copy(w_hbm.at[pl.ds(0, CROWS1), :], buf_a, sem_a).wait()
    pltpu.sync_copy(hist, out_hbm.at[wid])


@functools.cache
def _hist_pass2_kernel():
    return pl.kernel(
        _hist_pass2_body,
        out_type=jax.ShapeDtypeStruct((NW, B2), jnp.int32),
        mesh=_mesh(),
        scratch_types=[
            pltpu.VMEM((CROWS2, IN_F), jnp.float32),
            pltpu.VMEM((CROWS2, IN_F), jnp.float32),
            pltpu.VMEM((B2,), jnp.int32),
            pltpu.VMEM((L,), jnp.int32),
            pltpu.SemaphoreType.DMA,
            pltpu.SemaphoreType.DMA,
        ],
        compiler_params=pltpu.CompilerParams(needs_layout_passes=False),
    )


def _hist_pass2_body(w_hbm, b0_hbm, out_hbm, buf_a, buf_b, hist, b0_v,
                     sem_a, sem_b):
    wid = _worker_id()
    row0 = wid * ROWS_W
    _zero_hist(hist, B2)
    pltpu.sync_copy(b0_hbm, b0_v)
    b0 = b0_v[...]
    ones = jnp.ones((L,), jnp.int32)

    def start(g, buf, sem):
        g = jnp.minimum(g, NCH2 - 1)
        pltpu.async_copy(w_hbm.at[pl.ds(row0 + g * CROWS2, CROWS2), :], buf, sem)

    def process(buf):
        @plsc.parallel_loop(0, CROWS2 * VPC, unroll=UNROLL)
        def vec_body(j):
            r = lax.shift_right_logical(j, 8)
            c = j & (VPC - 1)
            p = plsc.bitcast(jnp.abs(buf[r, pl.ds(c * L, L)]), jnp.int32)
            top = lax.shift_right_logical(p, 16)
            low = p & 0xFFFF
            plsc.addupdate_scatter(hist, [low], ones, mask=top == b0)

    start(0, buf_a, sem_a)

    def pair_body(g2, _):
        start(2 * g2 + 1, buf_b, sem_b)
        pltpu.make_async_copy(w_hbm.at[pl.ds(0, CROWS2), :], buf_a, sem_a).wait()
        process(buf_a)
        start(2 * g2 + 2, buf_a, sem_a)
        pltpu.make_async_copy(w_hbm.at[pl.ds(0, CROWS2), :], buf_b, sem_b).wait()
        process(buf_b)
        return 0

    lax.fori_loop(0, NCH2 // 2, pair_body, 0)
    pltpu.make_async_copy(w_hbm.at[pl.ds(0, CROWS2), :], buf_a, sem_a).wait()
    pltpu.sync_copy(hist, out_hbm.at[wid])


BM = 2048
BN = 512


def _mm_body(t_ref, x_ref, w_ref, b_ref, o_ref):
    w = w_ref[...]
    t = t_ref[0, 0]
    w_eff = jnp.where(jnp.abs(w) >= t, w, 0.0).astype(jnp.bfloat16)
    acc = lax.dot_general(
        x_ref[...], w_eff, (((1,), (1,)), ((), ())),
        preferred_element_type=jnp.float32,
    )
    o_ref[...] = acc + b_ref[...]


def _masked_matmul(inputs, weight, bias, threshold):
    m_tokens = inputs.shape[0]
    grid = (m_tokens // BM, OUT_F // BN)
    return pl.pallas_call(
        _mm_body,
        grid=grid,
        in_specs=[
            pl.BlockSpec(memory_space=pltpu.SMEM),
            pl.BlockSpec((BM, IN_F), lambda m, n: (m, 0)),
            pl.BlockSpec((BN, IN_F), lambda m, n: (n, 0)),
            pl.BlockSpec((1, BN), lambda m, n: (0, n)),
        ],
        out_specs=pl.BlockSpec((BM, BN), lambda m, n: (m, n)),
        out_shape=jax.ShapeDtypeStruct((m_tokens, OUT_F), jnp.float32),
        compiler_params=pltpu.CompilerParams(
            dimension_semantics=("parallel", "parallel"),
            vmem_limit_bytes=100 * 1024 * 1024,
        ),
    )(threshold.reshape(1, 1), inputs.astype(jnp.bfloat16), weight,
      bias.reshape(1, OUT_F))


def _hist_pass2(w_flat, b0_vec):
    return _hist_pass2_kernel()(w_flat, b0_vec)


def _threshold_from_hists(w_flat, h1_rows):
    h1 = jnp.sum(h1_rows, axis=0)
    c1 = jnp.cumsum(h1)
    b0 = jnp.searchsorted(c1, RANK, side="right").astype(jnp.int32)
    below = jnp.where(b0 > 0, c1[jnp.maximum(b0 - 1, 0)], 0)
    r0 = RANK - below
    h2_rows = _hist_pass2(w_flat, jnp.full((L,), b0, jnp.int32))
    h2 = jnp.sum(h2_rows, axis=0)
    c2 = jnp.cumsum(h2)
    low = jnp.searchsorted(c2, r0, side="right").astype(jnp.int32)
    pattern = (b0 << 16) | low
    return lax.bitcast_convert_type(pattern, jnp.float32)


def kernel(inputs, weight, bias):
    h1_rows = _hist_pass1_kernel()(weight)
    threshold = _threshold_from_hists(weight, h1_rows)
    return _masked_matmul(inputs, weight, bias, threshold)
